# Initial kernel scaffold; baseline (speedup 1.0000x reference)
import jax
import jax.numpy as jnp
from jax.experimental import pallas as pl


def kernel(x, edge_filter, weight, edge_src, edge_dst):
    raise NotImplementedError("write your pallas kernel here")


# speedup vs baseline: 5.0022x; 5.0022x over previous
import functools

import jax
import jax.numpy as jnp
from jax import lax
from jax.experimental import pallas as pl
from jax.experimental.pallas import tpu as pltpu
from jax.experimental.pallas import tpu_sc as plsc

N = 10000
E = 320000
D = 128
F = 16

NC = 2
NS = 16
NW = NC * NS

G = 128
GROUPS_PER_TILE = 79
EPAD = NW * GROUPS_PER_TILE * G
NPAD = 10240
ROWS_PER_SUB = NPAD // NS
ZCHUNK = 128
LANES = 16


def _scale_body(f_ref, w_ref, o_ref):
    o_ref[...] = jnp.sum(f_ref[...], axis=-1) * jnp.sum(w_ref[...], axis=-1)


def _compute_scale(edge_filter, weight):
    BR = 200
    return pl.pallas_call(
        _scale_body,
        grid=(4000 // BR,),
        in_specs=[
            pl.BlockSpec((BR, 80, F), lambda i: (i, 0, 0)),
            pl.BlockSpec((BR, 80, F), lambda i: (i, 0, 0)),
        ],
        out_specs=pl.BlockSpec((BR, 80), lambda i: (i, 0)),
        out_shape=jax.ShapeDtypeStruct((4000, 80), jnp.float32),
    )(edge_filter, weight)


def _add_body(p_ref, o_ref):
    o_ref[...] = p_ref[0] + p_ref[1]


def _combine(partial):
    BR = 2000
    return pl.pallas_call(
        _add_body,
        grid=(N // BR,),
        in_specs=[pl.BlockSpec((NC, BR, D), lambda i: (0, i, 0))],
        out_specs=pl.BlockSpec((BR, D), lambda i: (i, 0)),
        out_shape=jax.ShapeDtypeStruct((N, D), jnp.float32),
    )(partial)


def _sc_main(x, scale3d, src3d, dst3d):
    mesh = plsc.VectorSubcoreMesh(core_axis_name="c", subcore_axis_name="s")

    @functools.partial(
        pl.kernel,
        out_type=jax.ShapeDtypeStruct((NC, NPAD, D), jnp.float32),
        mesh=mesh,
        scratch_types=[
            pltpu.VMEM((GROUPS_PER_TILE, G), jnp.int32),
            pltpu.VMEM((GROUPS_PER_TILE, G), jnp.int32),
            pltpu.VMEM((GROUPS_PER_TILE, G), jnp.float32),
            pltpu.VMEM((G, D), jnp.float32),
            pltpu.VMEM_SHARED((NPAD, D), jnp.float32),
            pltpu.SemaphoreType.DMA,
        ],
    )
    def body(x_hbm, scale_hbm, src_hbm, dst_hbm, out_hbm,
             src_v, dst_v, scale_v, rows_v, acc, sem):
        cid = lax.axis_index("c")
        sid = lax.axis_index("s")
        wid = cid * NS + sid

        pltpu.sync_copy(src_hbm.at[wid], src_v)
        pltpu.sync_copy(dst_hbm.at[wid], dst_v)
        pltpu.sync_copy(scale_hbm.at[wid], scale_v)

        zero = jnp.zeros((LANES,), jnp.float32)

        def zrow(i, carry):
            for q in range(D // LANES):
                rows_v[i, pl.ds(q * LANES, LANES)] = zero
            return carry

        lax.fori_loop(0, ZCHUNK, zrow, 0)
        for k in range(ROWS_PER_SUB // ZCHUNK):
            pltpu.sync_copy(
                rows_v, acc.at[pl.ds(sid * ROWS_PER_SUB + k * ZCHUNK, ZCHUNK)])
        plsc.subcore_barrier()

        def jbody(j, carry):
            pltpu.async_copy(x_hbm.at[src_v.at[j]], rows_v, sem).wait()
            for g in range(G // LANES):
                s16 = scale_v[j, pl.ds(g * LANES, LANES)]
                for t in range(LANES):
                    e = g * LANES + t
                    s = s16[t]
                    for q in range(D // LANES):
                        sl = pl.ds(q * LANES, LANES)
                        rows_v[e, sl] = rows_v[e, sl] * s
            pltpu.sync_copy(rows_v, acc.at[dst_v.at[j]], add=True)
            return carry

        lax.fori_loop(0, GROUPS_PER_TILE, jbody, 0)
        plsc.subcore_barrier()

        for k in range(ROWS_PER_SUB // ZCHUNK):
            r0 = sid * ROWS_PER_SUB + k * ZCHUNK
            pltpu.sync_copy(acc.at[pl.ds(r0, ZCHUNK)],
                            out_hbm.at[cid, pl.ds(r0, ZCHUNK)])

    return body(x, scale3d, src3d, dst3d)


def kernel(x, edge_filter, weight, edge_src, edge_dst):
    scale = _compute_scale(
        edge_filter.reshape(4000, 80, F), weight.reshape(4000, 80, F))
    npad = EPAD - E
    pad_idx = (jnp.arange(npad, dtype=jnp.int32) * 29) % N
    scale3d = jnp.concatenate(
        [scale.reshape(E), jnp.zeros((npad,), jnp.float32)]
    ).reshape(NW, GROUPS_PER_TILE, G)
    src3d = jnp.concatenate(
        [edge_src.astype(jnp.int32), pad_idx]).reshape(NW, GROUPS_PER_TILE, G)
    dst3d = jnp.concatenate(
        [edge_dst.astype(jnp.int32), pad_idx]).reshape(NW, GROUPS_PER_TILE, G)
    partial = _sc_main(x, scale3d, src3d, dst3d)
    return _combine(partial)
